# Initial kernel scaffold; baseline (speedup 1.0000x reference)
import jax
import jax.numpy as jnp
from jax.experimental import pallas as pl


def kernel(xyz, points, W1, b1, g1, be1, W2, b2, g2, be2, W3, b3, g3, be3):
    raise NotImplementedError("write your pallas kernel here")


# speedup vs baseline: 8.8081x; 8.8081x over previous
import dataclasses
import functools

import jax
import jax.numpy as jnp
from jax import lax
from jax.experimental import pallas as pl
from jax.experimental.pallas import tpu as pltpu
from jax.experimental.pallas import tpu_sc as plsc

B, N, S, K, D = 16, 2048, 512, 32, 64
R2 = 0.2 * 0.2
ROWS = B * S * K
BLK = 2048
NBLK = ROWS // BLK


def _fps_body(xt_ref, f0_ref, nx_ref, ny_ref, nz_ref, dist_ref):
    lane = lax.broadcasted_iota(jnp.int32, (B, N), 1)
    x = xt_ref[0]
    y = xt_ref[1]
    z = xt_ref[2]
    dist_ref[...] = jnp.full((B, N), 1e10, jnp.float32)
    sub = lax.broadcasted_iota(jnp.int32, (B, B), 0)
    ln2 = lax.broadcasted_iota(jnp.int32, (B, B), 1)
    eye_b = sub == ln2

    def step(s, fa):
        oh = lane == fa
        cx = jnp.sum(jnp.where(oh, x, 0.0), 1, keepdims=True)
        cy = jnp.sum(jnp.where(oh, y, 0.0), 1, keepdims=True)
        cz = jnp.sum(jnp.where(oh, z, 0.0), 1, keepdims=True)
        nx_ref[pl.ds(s, 1), :] = jnp.sum(jnp.where(eye_b, cx, 0.0), 0, keepdims=True)
        ny_ref[pl.ds(s, 1), :] = jnp.sum(jnp.where(eye_b, cy, 0.0), 0, keepdims=True)
        nz_ref[pl.ds(s, 1), :] = jnp.sum(jnp.where(eye_b, cz, 0.0), 0, keepdims=True)
        d = (x - cx) ** 2 + (y - cy) ** 2 + (z - cz) ** 2
        dist = jnp.minimum(dist_ref[...], d)
        dist_ref[...] = dist
        m = jnp.max(dist, 1, keepdims=True)
        return jnp.min(jnp.where(dist == m, lane, N), 1, keepdims=True)

    lax.fori_loop(0, S, step, f0_ref[...])


def _fps(xt, f0):
    return pl.pallas_call(
        _fps_body,
        out_shape=[jax.ShapeDtypeStruct((S, B), jnp.float32)] * 3,
        scratch_shapes=[pltpu.VMEM((B, N), jnp.float32)],
    )(xt, f0)


def _bq_body(nxyz_ref, xyzp_ref, sq_ref):
    nb = nxyz_ref[0]
    xp = xyzp_ref[0]
    n2s = (nb[:, 0:1] * nb[:, 0:1] + nb[:, 1:2] * nb[:, 1:2]
           + nb[:, 2:3] * nb[:, 2:3])
    n2p = (xp[0:1, :] * xp[0:1, :] + xp[1:2, :] * xp[1:2, :]
           + xp[2:3, :] * xp[2:3, :])
    dots = jnp.dot(nb, xp, preferred_element_type=jnp.float32)
    sq_ref[0] = (n2s + n2p) - 2.0 * dots


def _bq(nxyzp, xyzp):
    return pl.pallas_call(
        _bq_body,
        grid=(B,),
        in_specs=[pl.BlockSpec((1, S, 8), lambda b: (b, 0, 0)),
                  pl.BlockSpec((1, 8, N), lambda b: (b, 0, 0))],
        out_specs=pl.BlockSpec((1, S, N), lambda b: (b, 0, 0)),
        out_shape=jax.ShapeDtypeStruct((B, S, N), jnp.float32),
    )(nxyzp, xyzp)


def _sc_body(xs_hbm, ys_hbm, zs_hbm, nxyz_hbm, dsq_hbm, pts_hbm,
             gxn_hbm, gp_hbm,
             x_v, y_v, z_v, nx_v, ny_v, nz_v, dv_v,
             ibuf_v, pibuf_v, gxn_v, gp_v, sem, dsem, gsem):
    wid = lax.axis_index("c") * 16 + lax.axis_index("s")
    b = wid // 2
    off = (wid % 2) * 256
    iota = lax.broadcasted_iota(jnp.int32, (16,), 0)

    pltpu.async_copy(xs_hbm.at[pl.ds(b * N, N)], x_v, sem).wait()
    pltpu.async_copy(ys_hbm.at[pl.ds(b * N, N)], y_v, sem).wait()
    pltpu.async_copy(zs_hbm.at[pl.ds(b * N, N)], z_v, sem).wait()
    pltpu.async_copy(nxyz_hbm.at[pl.ds((b * 3 + 0) * S + off, 256)], nx_v, sem).wait()
    pltpu.async_copy(nxyz_hbm.at[pl.ds((b * 3 + 1) * S + off, 256)], ny_v, sem).wait()
    pltpu.async_copy(nxyz_hbm.at[pl.ds((b * 3 + 2) * S + off, 256)], nz_v, sem).wait()

    for i in range(16):
        gxn_v[pl.ds(i * 16, 16)] = jnp.zeros((16,), jnp.float32)

    srow0 = b * S + off

    @pl.loop(0, 256)
    def _j(j):
        sg = srow0 + j
        pltpu.async_copy(dsq_hbm.at[pl.ds(sg * N, N)], dv_v, dsem).wait()
        sel = (iota == (j % 16)).astype(jnp.float32)
        jc = (j // 16) * 16
        cx = jnp.sum(sel * nx_v[pl.ds(jc, 16)])
        cy = jnp.sum(sel * ny_v[pl.ds(jc, 16)])
        cz = jnp.sum(sel * nz_v[pl.ds(jc, 16)])

        def chunk(c, cnt):
            dcur = dv_v[pl.ds(c * 16, 16)]
            msk = dcur <= R2
            mi = msk.astype(jnp.int32)
            pos = cnt + plsc.cumsum(mi) - 1
            idxv = c * 16 + iota
            plsc.store_scatter(ibuf_v, [pos], idxv, mask=msk & (pos < 48))
            return cnt + jnp.sum(mi)

        cnt = lax.fori_loop(0, N // 16, chunk, jnp.int32(0))

        v0 = ibuf_v[pl.ds(0, 16)]
        fidx = jnp.sum(jnp.where(iota == 0, v0, 0))
        for half in (0, 1):
            slot = iota + 16 * half
            cur = ibuf_v[pl.ds(16 * half, 16)]
            ibuf_v[pl.ds(16 * half, 16)] = jnp.where(slot < cnt, cur, fidx)

        for half in (0, 1):
            giv = ibuf_v[pl.ds(16 * half, 16)]
            gx = plsc.load_gather(x_v, [giv]) - cx
            gy = plsc.load_gather(y_v, [giv]) - cy
            gz = plsc.load_gather(z_v, [giv]) - cz
            posb = (16 * half + iota) * 8
            plsc.store_scatter(gxn_v, [posb], gx)
            plsc.store_scatter(gxn_v, [posb + 1], gy)
            plsc.store_scatter(gxn_v, [posb + 2], gz)
            pibuf_v[pl.ds(16 * half, 16)] = giv + b * N

        row0 = sg * K
        pltpu.sync_copy(gxn_v, gxn_hbm.at[pl.ds(row0 * 8, 256)])
        pltpu.async_copy(pts_hbm.at[pibuf_v], gp_v, gsem).wait()
        pltpu.sync_copy(gp_v, gp_hbm.at[pl.ds(row0, K)])


def _sc_group(xs, ys, zs, nxyz_flat, dsq_flat, pts2):
    mesh = plsc.VectorSubcoreMesh(core_axis_name="c", subcore_axis_name="s")
    cp = pltpu.CompilerParams()
    if "needs_layout_passes" in pltpu.CompilerParams.__dataclass_fields__:
        cp = dataclasses.replace(cp, needs_layout_passes=False)
    kern = pl.kernel(
        _sc_body,
        compiler_params=cp,
        out_type=(
            jax.ShapeDtypeStruct((ROWS * 8,), jnp.float32),
            jax.ShapeDtypeStruct((ROWS, 128), jnp.float32),
        ),
        mesh=mesh,
        scratch_types=[
            pltpu.VMEM((N,), jnp.float32),
            pltpu.VMEM((N,), jnp.float32),
            pltpu.VMEM((N,), jnp.float32),
            pltpu.VMEM((256,), jnp.float32),
            pltpu.VMEM((256,), jnp.float32),
            pltpu.VMEM((256,), jnp.float32),
            pltpu.VMEM((N,), jnp.float32),
            pltpu.VMEM((64,), jnp.int32),
            pltpu.VMEM((K,), jnp.int32),
            pltpu.VMEM((256,), jnp.float32),
            pltpu.VMEM((K, 128), jnp.float32),
            pltpu.SemaphoreType.DMA,
            pltpu.SemaphoreType.DMA,
            pltpu.SemaphoreType.DMA,
        ],
    )
    return kern(xs, ys, zs, nxyz_flat, dsq_flat, pts2)


def _m1_body(gx_ref, gp_ref, wx_ref, wp_ref, b_ref, x1_ref, st_ref):
    i = pl.program_id(0)
    x1 = (jnp.dot(gx_ref[...], wx_ref[...], preferred_element_type=jnp.float32)
          + jnp.dot(gp_ref[...], wp_ref[...], preferred_element_type=jnp.float32)
          + b_ref[...])
    x1_ref[...] = x1

    @pl.when(i == 0)
    def _():
        st_ref[...] = jnp.zeros_like(st_ref)

    st_ref[0:1, :] += jnp.sum(x1, 0, keepdims=True)
    st_ref[1:2, :] += jnp.sum(x1 * x1, 0, keepdims=True)


def _mid_body(xin_ref, a_ref, c_ref, w_ref, b_ref, xo_ref, st_ref):
    i = pl.program_id(0)
    yprev = jnp.maximum(a_ref[...] * xin_ref[...] + c_ref[...], 0.0)
    xo = jnp.dot(yprev, w_ref[...], preferred_element_type=jnp.float32) + b_ref[...]
    xo_ref[...] = xo

    @pl.when(i == 0)
    def _():
        st_ref[...] = jnp.zeros_like(st_ref)

    st_ref[0:1, :] += jnp.sum(xo, 0, keepdims=True)
    st_ref[1:2, :] += jnp.sum(xo * xo, 0, keepdims=True)


def _m4_body(x3_ref, a_ref, c_ref, o_ref):
    y3 = jnp.maximum(a_ref[...] * x3_ref[...] + c_ref[...], 0.0)
    o_ref[...] = jnp.max(y3.reshape(BLK // K, K, 256), axis=1)


def _affine(st, g, be):
    m = st[0] / ROWS
    v = st[1] / ROWS - m * m
    a = g / jnp.sqrt(v + 1e-5)
    return a[None, :], (be - a * m)[None, :]


def _row_spec(cols):
    return pl.BlockSpec((BLK, cols), lambda i: (i, 0))


def _full_spec(r, c):
    return pl.BlockSpec((r, c), lambda i: (0, 0))


def _m1(gxn, gp, wx, wp, b1):
    return pl.pallas_call(
        _m1_body,
        grid=(NBLK,),
        in_specs=[_row_spec(8), _row_spec(128), _full_spec(8, 128),
                  _full_spec(128, 128), _full_spec(1, 128)],
        out_specs=[_row_spec(128), _full_spec(2, 128)],
        out_shape=[jax.ShapeDtypeStruct((ROWS, 128), jnp.float32),
                   jax.ShapeDtypeStruct((2, 128), jnp.float32)],
    )(gxn, gp, wx, wp, b1)


def _mid(xin, a, c, w, b, cout):
    cin = xin.shape[1]
    return pl.pallas_call(
        _mid_body,
        grid=(NBLK,),
        in_specs=[_row_spec(cin), _full_spec(1, cin), _full_spec(1, cin),
                  _full_spec(cin, cout), _full_spec(1, cout)],
        out_specs=[_row_spec(cout), _full_spec(2, cout)],
        out_shape=[jax.ShapeDtypeStruct((ROWS, cout), jnp.float32),
                   jax.ShapeDtypeStruct((2, cout), jnp.float32)],
    )(xin, a, c, w, b)


def _m4(x3, a, c):
    return pl.pallas_call(
        _m4_body,
        grid=(NBLK,),
        in_specs=[_row_spec(256), _full_spec(1, 256), _full_spec(1, 256)],
        out_specs=pl.BlockSpec((BLK // K, 256), lambda i: (i, 0)),
        out_shape=jax.ShapeDtypeStruct((B * S, 256), jnp.float32),
    )(x3, a, c)


def kernel(xyz, points, W1, b1, g1, be1, W2, b2, g2, be2, W3, b3, g3, be3):
    xt = jnp.transpose(xyz, (2, 0, 1))
    f0 = jax.random.randint(jax.random.key(42), (B,), 0, N, jnp.int32)[:, None]
    nx, ny, nz = _fps(xt, f0)
    nxyz_flat = jnp.concatenate(
        [nx.T[:, None, :], ny.T[:, None, :], nz.T[:, None, :]], 1).reshape(-1)
    new_xyz = jnp.stack([nx.T, ny.T, nz.T], -1)

    nxyzp = jnp.pad(new_xyz, ((0, 0), (0, 0), (0, 5)))
    xyzp = jnp.pad(jnp.transpose(xyz, (0, 2, 1)), ((0, 0), (0, 5), (0, 0)))
    dsq = _bq(nxyzp, xyzp).reshape(-1)

    pts2 = jnp.pad(points.reshape(B * N, D), ((0, 0), (0, 128 - D)))
    gxn_flat, gp = _sc_group(
        xyz[:, :, 0].reshape(-1), xyz[:, :, 1].reshape(-1),
        xyz[:, :, 2].reshape(-1), nxyz_flat, dsq, pts2)
    gxn = gxn_flat.reshape(ROWS, 8)

    wx = jnp.pad(W1[:, :3].T, ((0, 5), (0, 0)))
    wp = jnp.pad(W1[:, 3:].T, ((0, 128 - D), (0, 0)))
    x1, st1 = _m1(gxn, gp, wx, wp, b1[None, :])
    a1, c1 = _affine(st1, g1, be1)
    x2, st2 = _mid(x1, a1, c1, W2.T, b2[None, :], 128)
    a2, c2 = _affine(st2, g2, be2)
    x3, st3 = _mid(x2, a2, c2, W3.T, b3[None, :], 256)
    a3, c3 = _affine(st3, g3, be3)
    out = _m4(x3, a3, c3)

    return new_xyz, out.reshape(B, S, 256)


# speedup vs baseline: 10.1221x; 1.1492x over previous
import dataclasses
import functools

import jax
import jax.numpy as jnp
from jax import lax
from jax.experimental import pallas as pl
from jax.experimental.pallas import tpu as pltpu
from jax.experimental.pallas import tpu_sc as plsc

B, N, S, K, D = 16, 2048, 512, 32, 64
R2 = 0.2 * 0.2
ROWS = B * S * K
BLK = 2048
NBLK = ROWS // BLK


def _fps_body(xt_ref, f0_ref, nx_ref, ny_ref, nz_ref, dist_ref):
    lane = lax.broadcasted_iota(jnp.int32, (B, N), 1)
    x = xt_ref[0]
    y = xt_ref[1]
    z = xt_ref[2]
    dist_ref[...] = jnp.full((B, N), 1e10, jnp.float32)
    sub = lax.broadcasted_iota(jnp.int32, (B, B), 0)
    ln2 = lax.broadcasted_iota(jnp.int32, (B, B), 1)
    eye_b = sub == ln2

    def step(s, fa):
        oh = lane == fa
        cx = jnp.sum(jnp.where(oh, x, 0.0), 1, keepdims=True)
        cy = jnp.sum(jnp.where(oh, y, 0.0), 1, keepdims=True)
        cz = jnp.sum(jnp.where(oh, z, 0.0), 1, keepdims=True)
        nx_ref[pl.ds(s, 1), :] = jnp.sum(jnp.where(eye_b, cx, 0.0), 0, keepdims=True)
        ny_ref[pl.ds(s, 1), :] = jnp.sum(jnp.where(eye_b, cy, 0.0), 0, keepdims=True)
        nz_ref[pl.ds(s, 1), :] = jnp.sum(jnp.where(eye_b, cz, 0.0), 0, keepdims=True)
        d = (x - cx) ** 2 + (y - cy) ** 2 + (z - cz) ** 2
        dist = jnp.minimum(dist_ref[...], d)
        dist_ref[...] = dist
        m = jnp.max(dist, 1, keepdims=True)
        return jnp.min(jnp.where(dist == m, lane, N), 1, keepdims=True)

    lax.fori_loop(0, S, step, f0_ref[...])


def _fps(xt, f0):
    return pl.pallas_call(
        _fps_body,
        out_shape=[jax.ShapeDtypeStruct((S, B), jnp.float32)] * 3,
        scratch_shapes=[pltpu.VMEM((B, N), jnp.float32)],
    )(xt, f0)


def _bq_body(nxyz_ref, xyzp_ref, sq_ref):
    nb = nxyz_ref[0]
    xp = xyzp_ref[0]
    n2s = (nb[:, 0:1] * nb[:, 0:1] + nb[:, 1:2] * nb[:, 1:2]
           + nb[:, 2:3] * nb[:, 2:3])
    n2p = (xp[0:1, :] * xp[0:1, :] + xp[1:2, :] * xp[1:2, :]
           + xp[2:3, :] * xp[2:3, :])
    dots = jnp.dot(nb, xp, preferred_element_type=jnp.float32)
    sq_ref[0] = (n2s + n2p) - 2.0 * dots


def _bq(nxyzp, xyzp):
    return pl.pallas_call(
        _bq_body,
        grid=(B,),
        in_specs=[pl.BlockSpec((1, S, 8), lambda b: (b, 0, 0)),
                  pl.BlockSpec((1, 8, N), lambda b: (b, 0, 0))],
        out_specs=pl.BlockSpec((1, S, N), lambda b: (b, 0, 0)),
        out_shape=jax.ShapeDtypeStruct((B, S, N), jnp.float32),
    )(nxyzp, xyzp)


GB = 8
NBLOCKS = 256 // GB


def _sc_body(xs_hbm, ys_hbm, zs_hbm, nxyz_hbm, dsq_hbm, pts_hbm,
             gxn_hbm, gp_hbm,
             x_v, y_v, z_v, nx_v, ny_v, nz_v, dv0_v, dv1_v,
             ibuf_v, pibuf_v, gxn_v, gp_v, sem, dsem, gsem):
    wid = lax.axis_index("c") * 16 + lax.axis_index("s")
    b = wid // 2
    off = (wid % 2) * 256
    iota = lax.broadcasted_iota(jnp.int32, (16,), 0)

    pltpu.async_copy(xs_hbm.at[pl.ds(b * N, N)], x_v, sem).wait()
    pltpu.async_copy(ys_hbm.at[pl.ds(b * N, N)], y_v, sem).wait()
    pltpu.async_copy(zs_hbm.at[pl.ds(b * N, N)], z_v, sem).wait()
    pltpu.async_copy(nxyz_hbm.at[pl.ds((b * 3 + 0) * S + off, 256)], nx_v, sem).wait()
    pltpu.async_copy(nxyz_hbm.at[pl.ds((b * 3 + 1) * S + off, 256)], ny_v, sem).wait()
    pltpu.async_copy(nxyz_hbm.at[pl.ds((b * 3 + 2) * S + off, 256)], nz_v, sem).wait()

    @pl.loop(0, GB * K * 8 // 16)
    def _z(i):
        gxn_v[pl.ds(i * 16, 16)] = jnp.zeros((16,), jnp.float32)

    srow0 = b * S + off
    DB = GB * N

    def wait_dv(dv):
        pltpu.make_async_copy(dsq_hbm.at[pl.ds(0, DB)], dv, dsem).wait()

    def process(bb, dv):
        for j in range(GB):
            cidx = bb * GB + j
            sel = (iota == (cidx % 16)).astype(jnp.float32)
            jc = (cidx // 16) * 16
            cx = jnp.sum(sel * nx_v[pl.ds(jc, 16)])
            cy = jnp.sum(sel * ny_v[pl.ds(jc, 16)])
            cz = jnp.sum(sel * nz_v[pl.ds(jc, 16)])

            def cond(st):
                return (st[1] < K) & (st[0] < N // 16)

            def chunk(st):
                c, cnt = st
                dcur = dv[pl.ds(j * N + c * 16, 16)]
                msk = dcur <= R2
                mi = msk.astype(jnp.int32)
                pos = cnt + plsc.cumsum(mi) - 1
                idxv = c * 16 + iota
                plsc.store_scatter(ibuf_v, [pos], idxv, mask=msk & (pos < 48))
                return c + 1, cnt + jnp.sum(mi)

            _, cnt = lax.while_loop(cond, chunk, (jnp.int32(0), jnp.int32(0)))

            v0 = ibuf_v[pl.ds(0, 16)]
            fidx = jnp.sum(jnp.where(iota == 0, v0, 0))
            for half in (0, 1):
                slot = iota + 16 * half
                cur = ibuf_v[pl.ds(16 * half, 16)]
                ibuf_v[pl.ds(16 * half, 16)] = jnp.where(slot < cnt, cur, fidx)

            for half in (0, 1):
                giv = ibuf_v[pl.ds(16 * half, 16)]
                gx = plsc.load_gather(x_v, [giv]) - cx
                gy = plsc.load_gather(y_v, [giv]) - cy
                gz = plsc.load_gather(z_v, [giv]) - cz
                posb = (j * K + 16 * half + iota) * 8
                plsc.store_scatter(gxn_v, [posb], gx)
                plsc.store_scatter(gxn_v, [posb + 1], gy)
                plsc.store_scatter(gxn_v, [posb + 2], gz)
                pibuf_v[pl.ds(j * K + 16 * half, 16)] = giv + b * N

        row0 = (srow0 + bb * GB) * K
        pltpu.sync_copy(gxn_v, gxn_hbm.at[pl.ds(row0 * 8, GB * K * 8)])
        h1 = pltpu.async_copy(pts_hbm.at[pibuf_v.at[pl.ds(0, 128)]],
                              gp_v.at[pl.ds(0, 128)], gsem)
        h2 = pltpu.async_copy(pts_hbm.at[pibuf_v.at[pl.ds(128, 128)]],
                              gp_v.at[pl.ds(128, 128)], gsem)
        h1.wait()
        h2.wait()
        pltpu.sync_copy(gp_v, gp_hbm.at[pl.ds(row0, GB * K)])

    pltpu.async_copy(dsq_hbm.at[pl.ds(srow0 * N, DB)], dv0_v, dsem)

    @pl.loop(0, NBLOCKS, step=2)
    def _blk(bi):
        pltpu.async_copy(dsq_hbm.at[pl.ds((srow0 + (bi + 1) * GB) * N, DB)],
                         dv1_v, dsem)
        wait_dv(dv0_v)
        process(bi, dv0_v)

        @pl.when(bi + 2 < NBLOCKS)
        def _():
            pltpu.async_copy(dsq_hbm.at[pl.ds((srow0 + (bi + 2) * GB) * N, DB)],
                             dv0_v, dsem)

        wait_dv(dv1_v)
        process(bi + 1, dv1_v)


def _sc_group(xs, ys, zs, nxyz_flat, dsq_flat, pts2):
    mesh = plsc.VectorSubcoreMesh(core_axis_name="c", subcore_axis_name="s")
    cp = pltpu.CompilerParams()
    if "needs_layout_passes" in pltpu.CompilerParams.__dataclass_fields__:
        cp = dataclasses.replace(cp, needs_layout_passes=False)
    kern = pl.kernel(
        _sc_body,
        compiler_params=cp,
        out_type=(
            jax.ShapeDtypeStruct((ROWS * 8,), jnp.float32),
            jax.ShapeDtypeStruct((ROWS, 128), jnp.float32),
        ),
        mesh=mesh,
        scratch_types=[
            pltpu.VMEM((N,), jnp.float32),
            pltpu.VMEM((N,), jnp.float32),
            pltpu.VMEM((N,), jnp.float32),
            pltpu.VMEM((256,), jnp.float32),
            pltpu.VMEM((256,), jnp.float32),
            pltpu.VMEM((256,), jnp.float32),
            pltpu.VMEM((GB * N,), jnp.float32),
            pltpu.VMEM((GB * N,), jnp.float32),
            pltpu.VMEM((64,), jnp.int32),
            pltpu.VMEM((GB * K,), jnp.int32),
            pltpu.VMEM((GB * K * 8,), jnp.float32),
            pltpu.VMEM((GB * K, 128), jnp.float32),
            pltpu.SemaphoreType.DMA,
            pltpu.SemaphoreType.DMA,
            pltpu.SemaphoreType.DMA,
        ],
    )
    return kern(xs, ys, zs, nxyz_flat, dsq_flat, pts2)


def _m1_body(gx_ref, gp_ref, wx_ref, wp_ref, b_ref, x1_ref, st_ref):
    i = pl.program_id(0)
    x1 = (jnp.dot(gx_ref[...], wx_ref[...], preferred_element_type=jnp.float32)
          + jnp.dot(gp_ref[...], wp_ref[...], preferred_element_type=jnp.float32)
          + b_ref[...])
    x1_ref[...] = x1

    @pl.when(i == 0)
    def _():
        st_ref[...] = jnp.zeros_like(st_ref)

    st_ref[0:1, :] += jnp.sum(x1, 0, keepdims=True)
    st_ref[1:2, :] += jnp.sum(x1 * x1, 0, keepdims=True)


def _mid_body(xin_ref, a_ref, c_ref, w_ref, b_ref, xo_ref, st_ref):
    i = pl.program_id(0)
    yprev = jnp.maximum(a_ref[...] * xin_ref[...] + c_ref[...], 0.0)
    xo = jnp.dot(yprev, w_ref[...], preferred_element_type=jnp.float32) + b_ref[...]
    xo_ref[...] = xo

    @pl.when(i == 0)
    def _():
        st_ref[...] = jnp.zeros_like(st_ref)

    st_ref[0:1, :] += jnp.sum(xo, 0, keepdims=True)
    st_ref[1:2, :] += jnp.sum(xo * xo, 0, keepdims=True)


def _m4_body(x3_ref, a_ref, c_ref, o_ref):
    y3 = jnp.maximum(a_ref[...] * x3_ref[...] + c_ref[...], 0.0)
    o_ref[...] = jnp.max(y3.reshape(BLK // K, K, 256), axis=1)


def _affine(st, g, be):
    m = st[0] / ROWS
    v = st[1] / ROWS - m * m
    a = g / jnp.sqrt(v + 1e-5)
    return a[None, :], (be - a * m)[None, :]


def _row_spec(cols):
    return pl.BlockSpec((BLK, cols), lambda i: (i, 0))


def _full_spec(r, c):
    return pl.BlockSpec((r, c), lambda i: (0, 0))


def _m1(gxn, gp, wx, wp, b1):
    return pl.pallas_call(
        _m1_body,
        grid=(NBLK,),
        in_specs=[_row_spec(8), _row_spec(128), _full_spec(8, 128),
                  _full_spec(128, 128), _full_spec(1, 128)],
        out_specs=[_row_spec(128), _full_spec(2, 128)],
        out_shape=[jax.ShapeDtypeStruct((ROWS, 128), jnp.float32),
                   jax.ShapeDtypeStruct((2, 128), jnp.float32)],
    )(gxn, gp, wx, wp, b1)


def _mid(xin, a, c, w, b, cout):
    cin = xin.shape[1]
    return pl.pallas_call(
        _mid_body,
        grid=(NBLK,),
        in_specs=[_row_spec(cin), _full_spec(1, cin), _full_spec(1, cin),
                  _full_spec(cin, cout), _full_spec(1, cout)],
        out_specs=[_row_spec(cout), _full_spec(2, cout)],
        out_shape=[jax.ShapeDtypeStruct((ROWS, cout), jnp.float32),
                   jax.ShapeDtypeStruct((2, cout), jnp.float32)],
    )(xin, a, c, w, b)


def _m4(x3, a, c):
    return pl.pallas_call(
        _m4_body,
        grid=(NBLK,),
        in_specs=[_row_spec(256), _full_spec(1, 256), _full_spec(1, 256)],
        out_specs=pl.BlockSpec((BLK // K, 256), lambda i: (i, 0)),
        out_shape=jax.ShapeDtypeStruct((B * S, 256), jnp.float32),
    )(x3, a, c)


def kernel(xyz, points, W1, b1, g1, be1, W2, b2, g2, be2, W3, b3, g3, be3):
    xt = jnp.transpose(xyz, (2, 0, 1))
    f0 = jax.random.randint(jax.random.key(42), (B,), 0, N, jnp.int32)[:, None]
    nx, ny, nz = _fps(xt, f0)
    nxyz_flat = jnp.concatenate(
        [nx.T[:, None, :], ny.T[:, None, :], nz.T[:, None, :]], 1).reshape(-1)
    new_xyz = jnp.stack([nx.T, ny.T, nz.T], -1)

    nxyzp = jnp.pad(new_xyz, ((0, 0), (0, 0), (0, 5)))
    xyzp = jnp.pad(jnp.transpose(xyz, (0, 2, 1)), ((0, 0), (0, 5), (0, 0)))
    dsq = _bq(nxyzp, xyzp).reshape(-1)

    pts2 = jnp.pad(points.reshape(B * N, D), ((0, 0), (0, 128 - D)))
    gxn_flat, gp = _sc_group(
        xyz[:, :, 0].reshape(-1), xyz[:, :, 1].reshape(-1),
        xyz[:, :, 2].reshape(-1), nxyz_flat, dsq, pts2)
    gxn = gxn_flat.reshape(ROWS, 8)

    wx = jnp.pad(W1[:, :3].T, ((0, 5), (0, 0)))
    wp = jnp.pad(W1[:, 3:].T, ((0, 128 - D), (0, 0)))
    x1, st1 = _m1(gxn, gp, wx, wp, b1[None, :])
    a1, c1 = _affine(st1, g1, be1)
    x2, st2 = _mid(x1, a1, c1, W2.T, b2[None, :], 128)
    a2, c2 = _affine(st2, g2, be2)
    x3, st3 = _mid(x2, a2, c2, W3.T, b3[None, :], 256)
    a3, c3 = _affine(st3, g3, be3)
    out = _m4(x3, a3, c3)

    return new_xyz, out.reshape(B, S, 256)


# speedup vs baseline: 10.4988x; 1.0372x over previous
import dataclasses
import functools

import jax
import jax.numpy as jnp
from jax import lax
from jax.experimental import pallas as pl
from jax.experimental.pallas import tpu as pltpu
from jax.experimental.pallas import tpu_sc as plsc

B, N, S, K, D = 16, 2048, 512, 32, 64
R2 = 0.2 * 0.2
ROWS = B * S * K
BLK = 2048
NBLK = ROWS // BLK


def _fps_body(xt_ref, f0_ref, nx_ref, ny_ref, nz_ref, dist_ref):
    lane = lax.broadcasted_iota(jnp.int32, (B, N), 1)
    x = xt_ref[0]
    y = xt_ref[1]
    z = xt_ref[2]
    dist_ref[...] = jnp.full((B, N), 1e10, jnp.float32)
    sub = lax.broadcasted_iota(jnp.int32, (B, B), 0)
    ln2 = lax.broadcasted_iota(jnp.int32, (B, B), 1)
    eye_b = sub == ln2

    def step(s, fa):
        oh = lane == fa
        cx = jnp.sum(jnp.where(oh, x, 0.0), 1, keepdims=True)
        cy = jnp.sum(jnp.where(oh, y, 0.0), 1, keepdims=True)
        cz = jnp.sum(jnp.where(oh, z, 0.0), 1, keepdims=True)
        nx_ref[pl.ds(s, 1), :] = jnp.sum(jnp.where(eye_b, cx, 0.0), 0, keepdims=True)
        ny_ref[pl.ds(s, 1), :] = jnp.sum(jnp.where(eye_b, cy, 0.0), 0, keepdims=True)
        nz_ref[pl.ds(s, 1), :] = jnp.sum(jnp.where(eye_b, cz, 0.0), 0, keepdims=True)
        d = (x - cx) ** 2 + (y - cy) ** 2 + (z - cz) ** 2
        dist = jnp.minimum(dist_ref[...], d)
        dist_ref[...] = dist
        m = jnp.max(dist, 1, keepdims=True)
        return jnp.min(jnp.where(dist == m, lane, N), 1, keepdims=True)

    lax.fori_loop(0, S, step, f0_ref[...])


def _fps(xt, f0):
    return pl.pallas_call(
        _fps_body,
        out_shape=[jax.ShapeDtypeStruct((S, B), jnp.float32)] * 3,
        scratch_shapes=[pltpu.VMEM((B, N), jnp.float32)],
    )(xt, f0)


def _bq_body(nxyz_ref, xyzp_ref, sq_ref):
    nb = nxyz_ref[0]
    xp = xyzp_ref[0]
    n2s = (nb[:, 0:1] * nb[:, 0:1] + nb[:, 1:2] * nb[:, 1:2]
           + nb[:, 2:3] * nb[:, 2:3])
    n2p = (xp[0:1, :] * xp[0:1, :] + xp[1:2, :] * xp[1:2, :]
           + xp[2:3, :] * xp[2:3, :])
    dots = jnp.dot(nb, xp, preferred_element_type=jnp.float32)
    sq_ref[0] = (n2s + n2p) - 2.0 * dots


def _bq(nxyzp, xyzp):
    return pl.pallas_call(
        _bq_body,
        grid=(B,),
        in_specs=[pl.BlockSpec((1, S, 8), lambda b: (b, 0, 0)),
                  pl.BlockSpec((1, 8, N), lambda b: (b, 0, 0))],
        out_specs=pl.BlockSpec((1, S, N), lambda b: (b, 0, 0)),
        out_shape=jax.ShapeDtypeStruct((B, S, N), jnp.float32),
    )(nxyzp, xyzp)


GB = 8
NBLOCKS = 256 // GB


def _sc_body(xs_hbm, ys_hbm, zs_hbm, nxyz_hbm, dsq_hbm, pts_hbm,
             gxn_hbm, gp_hbm,
             x_v, y_v, z_v, nx_v, ny_v, nz_v, dv0_v, dv1_v,
             ibuf_v, pibuf_v, gxn_v, gp_v, sem, dsem, gsem):
    wid = lax.axis_index("c") * 16 + lax.axis_index("s")
    b = wid // 2
    off = (wid % 2) * 256
    iota = lax.broadcasted_iota(jnp.int32, (16,), 0)

    pltpu.async_copy(xs_hbm.at[pl.ds(b * N, N)], x_v, sem).wait()
    pltpu.async_copy(ys_hbm.at[pl.ds(b * N, N)], y_v, sem).wait()
    pltpu.async_copy(zs_hbm.at[pl.ds(b * N, N)], z_v, sem).wait()
    pltpu.async_copy(nxyz_hbm.at[pl.ds((b * 3 + 0) * S + off, 256)], nx_v, sem).wait()
    pltpu.async_copy(nxyz_hbm.at[pl.ds((b * 3 + 1) * S + off, 256)], ny_v, sem).wait()
    pltpu.async_copy(nxyz_hbm.at[pl.ds((b * 3 + 2) * S + off, 256)], nz_v, sem).wait()

    @pl.loop(0, GB * K * 8 // 16)
    def _z(i):
        gxn_v[pl.ds(i * 16, 16)] = jnp.zeros((16,), jnp.float32)

    srow0 = b * S + off
    DB = GB * N

    def wait_dv(dv):
        pltpu.make_async_copy(dsq_hbm.at[pl.ds(0, DB)], dv, dsem).wait()

    def process(bb, dv):
        for j in range(GB):
            cidx = bb * GB + j
            sel = (iota == (cidx % 16)).astype(jnp.float32)
            jc = (cidx // 16) * 16
            cx = jnp.sum(sel * nx_v[pl.ds(jc, 16)])
            cy = jnp.sum(sel * ny_v[pl.ds(jc, 16)])
            cz = jnp.sum(sel * nz_v[pl.ds(jc, 16)])

            def cond(st):
                return (st[1] < K) & (st[0] < N // 16)

            def chunk(st):
                c, cnt = st
                dcur = dv[pl.ds(j * N + c * 16, 16)]
                msk = dcur <= R2
                idxv = c * 16 + iota
                plsc.store_compressed(ibuf_v.at[pl.ds(cnt, 16)], idxv, mask=msk)
                pc = plsc.all_reduce_population_count(msk)
                return c + 1, cnt + pc[0]

            _, cnt = lax.while_loop(cond, chunk, (jnp.int32(0), jnp.int32(0)))

            v0 = ibuf_v[pl.ds(0, 16)]
            fidx = jnp.sum(jnp.where(iota == 0, v0, 0))
            for half in (0, 1):
                slot = iota + 16 * half
                cur = ibuf_v[pl.ds(16 * half, 16)]
                ibuf_v[pl.ds(16 * half, 16)] = jnp.where(slot < cnt, cur, fidx)

            for half in (0, 1):
                giv = ibuf_v[pl.ds(16 * half, 16)]
                gx = plsc.load_gather(x_v, [giv]) - cx
                gy = plsc.load_gather(y_v, [giv]) - cy
                gz = plsc.load_gather(z_v, [giv]) - cz
                posb = (j * K + 16 * half + iota) * 8
                plsc.store_scatter(gxn_v, [posb], gx)
                plsc.store_scatter(gxn_v, [posb + 1], gy)
                plsc.store_scatter(gxn_v, [posb + 2], gz)
                pibuf_v[pl.ds(j * K + 16 * half, 16)] = giv + b * N

        row0 = (srow0 + bb * GB) * K
        pltpu.sync_copy(gxn_v, gxn_hbm.at[pl.ds(row0 * 8, GB * K * 8)])
        h1 = pltpu.async_copy(pts_hbm.at[pibuf_v.at[pl.ds(0, 128)]],
                              gp_v.at[pl.ds(0, 128)], gsem)
        h2 = pltpu.async_copy(pts_hbm.at[pibuf_v.at[pl.ds(128, 128)]],
                              gp_v.at[pl.ds(128, 128)], gsem)
        h1.wait()
        h2.wait()
        pltpu.sync_copy(gp_v, gp_hbm.at[pl.ds(row0, GB * K)])

    pltpu.async_copy(dsq_hbm.at[pl.ds(srow0 * N, DB)], dv0_v, dsem)

    @pl.loop(0, NBLOCKS, step=2)
    def _blk(bi):
        pltpu.async_copy(dsq_hbm.at[pl.ds((srow0 + (bi + 1) * GB) * N, DB)],
                         dv1_v, dsem)
        wait_dv(dv0_v)
        process(bi, dv0_v)

        @pl.when(bi + 2 < NBLOCKS)
        def _():
            pltpu.async_copy(dsq_hbm.at[pl.ds((srow0 + (bi + 2) * GB) * N, DB)],
                             dv0_v, dsem)

        wait_dv(dv1_v)
        process(bi + 1, dv1_v)


def _sc_group(xs, ys, zs, nxyz_flat, dsq_flat, pts2):
    mesh = plsc.VectorSubcoreMesh(core_axis_name="c", subcore_axis_name="s")
    cp = pltpu.CompilerParams()
    if "needs_layout_passes" in pltpu.CompilerParams.__dataclass_fields__:
        cp = dataclasses.replace(cp, needs_layout_passes=False)
    kern = pl.kernel(
        _sc_body,
        compiler_params=cp,
        out_type=(
            jax.ShapeDtypeStruct((ROWS * 8,), jnp.float32),
            jax.ShapeDtypeStruct((ROWS, 128), jnp.float32),
        ),
        mesh=mesh,
        scratch_types=[
            pltpu.VMEM((N,), jnp.float32),
            pltpu.VMEM((N,), jnp.float32),
            pltpu.VMEM((N,), jnp.float32),
            pltpu.VMEM((256,), jnp.float32),
            pltpu.VMEM((256,), jnp.float32),
            pltpu.VMEM((256,), jnp.float32),
            pltpu.VMEM((GB * N,), jnp.float32),
            pltpu.VMEM((GB * N,), jnp.float32),
            pltpu.VMEM((64,), jnp.int32),
            pltpu.VMEM((GB * K,), jnp.int32),
            pltpu.VMEM((GB * K * 8,), jnp.float32),
            pltpu.VMEM((GB * K, 128), jnp.float32),
            pltpu.SemaphoreType.DMA,
            pltpu.SemaphoreType.DMA,
            pltpu.SemaphoreType.DMA,
        ],
    )
    return kern(xs, ys, zs, nxyz_flat, dsq_flat, pts2)


def _m1_body(gx_ref, gp_ref, wx_ref, wp_ref, b_ref, x1_ref, st_ref):
    i = pl.program_id(0)
    x1 = (jnp.dot(gx_ref[...], wx_ref[...], preferred_element_type=jnp.float32)
          + jnp.dot(gp_ref[...], wp_ref[...], preferred_element_type=jnp.float32)
          + b_ref[...])
    x1_ref[...] = x1

    @pl.when(i == 0)
    def _():
        st_ref[...] = jnp.zeros_like(st_ref)

    st_ref[0:1, :] += jnp.sum(x1, 0, keepdims=True)
    st_ref[1:2, :] += jnp.sum(x1 * x1, 0, keepdims=True)


def _mid_body(xin_ref, a_ref, c_ref, w_ref, b_ref, xo_ref, st_ref):
    i = pl.program_id(0)
    yprev = jnp.maximum(a_ref[...] * xin_ref[...] + c_ref[...], 0.0)
    xo = jnp.dot(yprev, w_ref[...], preferred_element_type=jnp.float32) + b_ref[...]
    xo_ref[...] = xo

    @pl.when(i == 0)
    def _():
        st_ref[...] = jnp.zeros_like(st_ref)

    st_ref[0:1, :] += jnp.sum(xo, 0, keepdims=True)
    st_ref[1:2, :] += jnp.sum(xo * xo, 0, keepdims=True)


def _m4_body(x3_ref, a_ref, c_ref, o_ref):
    y3 = jnp.maximum(a_ref[...] * x3_ref[...] + c_ref[...], 0.0)
    o_ref[...] = jnp.max(y3.reshape(BLK // K, K, 256), axis=1)


def _affine(st, g, be):
    m = st[0] / ROWS
    v = st[1] / ROWS - m * m
    a = g / jnp.sqrt(v + 1e-5)
    return a[None, :], (be - a * m)[None, :]


def _row_spec(cols):
    return pl.BlockSpec((BLK, cols), lambda i: (i, 0))


def _full_spec(r, c):
    return pl.BlockSpec((r, c), lambda i: (0, 0))


def _m1(gxn, gp, wx, wp, b1):
    return pl.pallas_call(
        _m1_body,
        grid=(NBLK,),
        in_specs=[_row_spec(8), _row_spec(128), _full_spec(8, 128),
                  _full_spec(128, 128), _full_spec(1, 128)],
        out_specs=[_row_spec(128), _full_spec(2, 128)],
        out_shape=[jax.ShapeDtypeStruct((ROWS, 128), jnp.float32),
                   jax.ShapeDtypeStruct((2, 128), jnp.float32)],
    )(gxn, gp, wx, wp, b1)


def _mid(xin, a, c, w, b, cout):
    cin = xin.shape[1]
    return pl.pallas_call(
        _mid_body,
        grid=(NBLK,),
        in_specs=[_row_spec(cin), _full_spec(1, cin), _full_spec(1, cin),
                  _full_spec(cin, cout), _full_spec(1, cout)],
        out_specs=[_row_spec(cout), _full_spec(2, cout)],
        out_shape=[jax.ShapeDtypeStruct((ROWS, cout), jnp.float32),
                   jax.ShapeDtypeStruct((2, cout), jnp.float32)],
    )(xin, a, c, w, b)


def _m4(x3, a, c):
    return pl.pallas_call(
        _m4_body,
        grid=(NBLK,),
        in_specs=[_row_spec(256), _full_spec(1, 256), _full_spec(1, 256)],
        out_specs=pl.BlockSpec((BLK // K, 256), lambda i: (i, 0)),
        out_shape=jax.ShapeDtypeStruct((B * S, 256), jnp.float32),
    )(x3, a, c)


def kernel(xyz, points, W1, b1, g1, be1, W2, b2, g2, be2, W3, b3, g3, be3):
    xt = jnp.transpose(xyz, (2, 0, 1))
    f0 = jax.random.randint(jax.random.key(42), (B,), 0, N, jnp.int32)[:, None]
    nx, ny, nz = _fps(xt, f0)
    nxyz_flat = jnp.concatenate(
        [nx.T[:, None, :], ny.T[:, None, :], nz.T[:, None, :]], 1).reshape(-1)
    new_xyz = jnp.stack([nx.T, ny.T, nz.T], -1)

    nxyzp = jnp.pad(new_xyz, ((0, 0), (0, 0), (0, 5)))
    xyzp = jnp.pad(jnp.transpose(xyz, (0, 2, 1)), ((0, 0), (0, 5), (0, 0)))
    dsq = _bq(nxyzp, xyzp).reshape(-1)

    pts2 = jnp.pad(points.reshape(B * N, D), ((0, 0), (0, 128 - D)))
    gxn_flat, gp = _sc_group(
        xyz[:, :, 0].reshape(-1), xyz[:, :, 1].reshape(-1),
        xyz[:, :, 2].reshape(-1), nxyz_flat, dsq, pts2)
    gxn = gxn_flat.reshape(ROWS, 8)

    wx = jnp.pad(W1[:, :3].T, ((0, 5), (0, 0)))
    wp = jnp.pad(W1[:, 3:].T, ((0, 128 - D), (0, 0)))
    x1, st1 = _m1(gxn, gp, wx, wp, b1[None, :])
    a1, c1 = _affine(st1, g1, be1)
    x2, st2 = _mid(x1, a1, c1, W2.T, b2[None, :], 128)
    a2, c2 = _affine(st2, g2, be2)
    x3, st3 = _mid(x2, a2, c2, W3.T, b3[None, :], 256)
    a3, c3 = _affine(st3, g3, be3)
    out = _m4(x3, a3, c3)

    return new_xyz, out.reshape(B, S, 256)


# speedup vs baseline: 12.5064x; 1.1912x over previous
import dataclasses
import functools

import jax
import jax.numpy as jnp
from jax import lax
from jax.experimental import pallas as pl
from jax.experimental.pallas import tpu as pltpu
from jax.experimental.pallas import tpu_sc as plsc

B, N, S, K, D = 16, 2048, 512, 32, 64
R2 = 0.2 * 0.2
ROWS = B * S * K
BLK = 4096
NBLK = ROWS // BLK


def _fps_body(xt_ref, f0_ref, nx_ref, ny_ref, nz_ref, dist_ref):
    lane = lax.broadcasted_iota(jnp.int32, (B, N), 1)
    x = xt_ref[0]
    y = xt_ref[1]
    z = xt_ref[2]
    dist_ref[...] = jnp.full((B, N), 1e10, jnp.float32)
    sub = lax.broadcasted_iota(jnp.int32, (B, B), 0)
    ln2 = lax.broadcasted_iota(jnp.int32, (B, B), 1)
    eye_b = sub == ln2

    def step(s, fa):
        oh = lane == fa
        cx = jnp.sum(jnp.where(oh, x, 0.0), 1, keepdims=True)
        cy = jnp.sum(jnp.where(oh, y, 0.0), 1, keepdims=True)
        cz = jnp.sum(jnp.where(oh, z, 0.0), 1, keepdims=True)
        nx_ref[pl.ds(s, 1), :] = jnp.sum(jnp.where(eye_b, cx, 0.0), 0, keepdims=True)
        ny_ref[pl.ds(s, 1), :] = jnp.sum(jnp.where(eye_b, cy, 0.0), 0, keepdims=True)
        nz_ref[pl.ds(s, 1), :] = jnp.sum(jnp.where(eye_b, cz, 0.0), 0, keepdims=True)
        d = (x - cx) ** 2 + (y - cy) ** 2 + (z - cz) ** 2
        dist = jnp.minimum(dist_ref[...], d)
        dist_ref[...] = dist
        m = jnp.max(dist, 1, keepdims=True)
        return jnp.min(jnp.where(dist == m, lane, N), 1, keepdims=True)

    lax.fori_loop(0, S, step, f0_ref[...])


def _fps(xt, f0):
    return pl.pallas_call(
        _fps_body,
        out_shape=[jax.ShapeDtypeStruct((S, B), jnp.float32)] * 3,
        scratch_shapes=[pltpu.VMEM((B, N), jnp.float32)],
    )(xt, f0)


def _bq_body(nxyz_ref, xyzp_ref, sq_ref):
    nb = nxyz_ref[0]
    xp = xyzp_ref[0]
    n2s = (nb[:, 0:1] * nb[:, 0:1] + nb[:, 1:2] * nb[:, 1:2]
           + nb[:, 2:3] * nb[:, 2:3])
    n2p = (xp[0:1, :] * xp[0:1, :] + xp[1:2, :] * xp[1:2, :]
           + xp[2:3, :] * xp[2:3, :])
    dots = jnp.dot(nb, xp, preferred_element_type=jnp.float32)
    sq_ref[0] = (n2s + n2p) - 2.0 * dots


def _bq(nxyzp, xyzp):
    return pl.pallas_call(
        _bq_body,
        grid=(B,),
        in_specs=[pl.BlockSpec((1, S, 8), lambda b: (b, 0, 0)),
                  pl.BlockSpec((1, 8, N), lambda b: (b, 0, 0))],
        out_specs=pl.BlockSpec((1, S, N), lambda b: (b, 0, 0)),
        out_shape=jax.ShapeDtypeStruct((B, S, N), jnp.float32),
    )(nxyzp, xyzp)


GB = 8
NBLOCKS = 256 // GB


def _sc_body(xs_hbm, ys_hbm, zs_hbm, nxyz_hbm, dsq_hbm, pts_hbm,
             gxn_hbm, gp_hbm,
             x_v, y_v, z_v, nx_v, ny_v, nz_v, dv0_v, dv1_v,
             ibuf_v, pibuf_v, gxn_v, gp_v, sem, dsem, gsem):
    wid = lax.axis_index("c") * 16 + lax.axis_index("s")
    b = wid // 2
    off = (wid % 2) * 256
    iota = lax.broadcasted_iota(jnp.int32, (16,), 0)

    pltpu.async_copy(xs_hbm.at[pl.ds(b * N, N)], x_v, sem).wait()
    pltpu.async_copy(ys_hbm.at[pl.ds(b * N, N)], y_v, sem).wait()
    pltpu.async_copy(zs_hbm.at[pl.ds(b * N, N)], z_v, sem).wait()
    pltpu.async_copy(nxyz_hbm.at[pl.ds((b * 3 + 0) * S + off, 256)], nx_v, sem).wait()
    pltpu.async_copy(nxyz_hbm.at[pl.ds((b * 3 + 1) * S + off, 256)], ny_v, sem).wait()
    pltpu.async_copy(nxyz_hbm.at[pl.ds((b * 3 + 2) * S + off, 256)], nz_v, sem).wait()

    @pl.loop(0, GB * K * 8 // 16)
    def _z(i):
        gxn_v[pl.ds(i * 16, 16)] = jnp.zeros((16,), jnp.float32)

    srow0 = b * S + off
    DB = GB * N

    def wait_dv(dv):
        pltpu.make_async_copy(dsq_hbm.at[pl.ds(0, DB)], dv, dsem).wait()

    def process(bb, dv):
        for j in range(GB):
            cidx = bb * GB + j
            sel = (iota == (cidx % 16)).astype(jnp.float32)
            jc = (cidx // 16) * 16
            cx = jnp.sum(sel * nx_v[pl.ds(jc, 16)])
            cy = jnp.sum(sel * ny_v[pl.ds(jc, 16)])
            cz = jnp.sum(sel * nz_v[pl.ds(jc, 16)])

            def cond(st):
                return (st[1] < K) & (st[0] < N // 16)

            def chunk(st):
                c, cnt = st
                dcur = dv[pl.ds(j * N + c * 16, 16)]
                msk = dcur <= R2
                idxv = c * 16 + iota
                plsc.store_compressed(ibuf_v.at[pl.ds(cnt, 16)], idxv, mask=msk)
                pc = plsc.all_reduce_population_count(msk)
                return c + 1, cnt + pc[0]

            _, cnt = lax.while_loop(cond, chunk, (jnp.int32(0), jnp.int32(0)))

            v0 = ibuf_v[pl.ds(0, 16)]
            fidx = jnp.sum(jnp.where(iota == 0, v0, 0))
            for half in (0, 1):
                slot = iota + 16 * half
                cur = ibuf_v[pl.ds(16 * half, 16)]
                ibuf_v[pl.ds(16 * half, 16)] = jnp.where(slot < cnt, cur, fidx)

            for half in (0, 1):
                giv = ibuf_v[pl.ds(16 * half, 16)]
                gx = plsc.load_gather(x_v, [giv]) - cx
                gy = plsc.load_gather(y_v, [giv]) - cy
                gz = plsc.load_gather(z_v, [giv]) - cz
                posb = (j * K + 16 * half + iota) * 8
                plsc.store_scatter(gxn_v, [posb], gx)
                plsc.store_scatter(gxn_v, [posb + 1], gy)
                plsc.store_scatter(gxn_v, [posb + 2], gz)
                pibuf_v[pl.ds(j * K + 16 * half, 16)] = giv + b * N

        row0 = (srow0 + bb * GB) * K
        pltpu.sync_copy(gxn_v, gxn_hbm.at[pl.ds(row0 * 8, GB * K * 8)])
        h1 = pltpu.async_copy(pts_hbm.at[pibuf_v.at[pl.ds(0, 128)]],
                              gp_v.at[pl.ds(0, 128)], gsem)
        h2 = pltpu.async_copy(pts_hbm.at[pibuf_v.at[pl.ds(128, 128)]],
                              gp_v.at[pl.ds(128, 128)], gsem)
        h1.wait()
        h2.wait()
        pltpu.sync_copy(gp_v, gp_hbm.at[pl.ds(row0, GB * K)])

    pltpu.async_copy(dsq_hbm.at[pl.ds(srow0 * N, DB)], dv0_v, dsem)

    @pl.loop(0, NBLOCKS, step=2)
    def _blk(bi):
        pltpu.async_copy(dsq_hbm.at[pl.ds((srow0 + (bi + 1) * GB) * N, DB)],
                         dv1_v, dsem)
        wait_dv(dv0_v)
        process(bi, dv0_v)

        @pl.when(bi + 2 < NBLOCKS)
        def _():
            pltpu.async_copy(dsq_hbm.at[pl.ds((srow0 + (bi + 2) * GB) * N, DB)],
                             dv0_v, dsem)

        wait_dv(dv1_v)
        process(bi + 1, dv1_v)


def _sc_group(xs, ys, zs, nxyz_flat, dsq_flat, pts2):
    mesh = plsc.VectorSubcoreMesh(core_axis_name="c", subcore_axis_name="s")
    cp = pltpu.CompilerParams()
    if "needs_layout_passes" in pltpu.CompilerParams.__dataclass_fields__:
        cp = dataclasses.replace(cp, needs_layout_passes=False)
    kern = pl.kernel(
        _sc_body,
        compiler_params=cp,
        out_type=(
            jax.ShapeDtypeStruct((ROWS * 8,), jnp.float32),
            jax.ShapeDtypeStruct((ROWS, 128), jnp.float32),
        ),
        mesh=mesh,
        scratch_types=[
            pltpu.VMEM((N,), jnp.float32),
            pltpu.VMEM((N,), jnp.float32),
            pltpu.VMEM((N,), jnp.float32),
            pltpu.VMEM((256,), jnp.float32),
            pltpu.VMEM((256,), jnp.float32),
            pltpu.VMEM((256,), jnp.float32),
            pltpu.VMEM((GB * N,), jnp.float32),
            pltpu.VMEM((GB * N,), jnp.float32),
            pltpu.VMEM((64,), jnp.int32),
            pltpu.VMEM((GB * K,), jnp.int32),
            pltpu.VMEM((GB * K * 8,), jnp.float32),
            pltpu.VMEM((GB * K, 128), jnp.float32),
            pltpu.SemaphoreType.DMA,
            pltpu.SemaphoreType.DMA,
            pltpu.SemaphoreType.DMA,
        ],
    )
    return kern(xs, ys, zs, nxyz_flat, dsq_flat, pts2)


def _m1_body(gx_ref, gp_ref, wx_ref, wp_ref, b_ref, x1_ref, st_ref):
    i = pl.program_id(0)
    x1 = (jnp.dot(gx_ref[...], wx_ref[...], preferred_element_type=jnp.float32)
          + jnp.dot(gp_ref[...], wp_ref[...], preferred_element_type=jnp.float32)
          + b_ref[...])
    x1_ref[...] = x1.astype(jnp.bfloat16)

    @pl.when(i == 0)
    def _():
        st_ref[...] = jnp.zeros_like(st_ref)

    st_ref[0:1, :] += jnp.sum(x1, 0, keepdims=True)
    st_ref[1:2, :] += jnp.sum(x1 * x1, 0, keepdims=True)


def _mid_body(xin_ref, a_ref, c_ref, w_ref, b_ref, xo_ref, st_ref):
    i = pl.program_id(0)
    xin = xin_ref[...].astype(jnp.float32)
    yprev = jnp.maximum(a_ref[...] * xin + c_ref[...], 0.0)
    xo = jnp.dot(yprev, w_ref[...], preferred_element_type=jnp.float32) + b_ref[...]
    xo_ref[...] = xo.astype(jnp.bfloat16)

    @pl.when(i == 0)
    def _():
        st_ref[...] = jnp.zeros_like(st_ref)

    st_ref[0:1, :] += jnp.sum(xo, 0, keepdims=True)
    st_ref[1:2, :] += jnp.sum(xo * xo, 0, keepdims=True)


def _m4_body(x3_ref, a_ref, c_ref, o_ref):
    y3 = jnp.maximum(a_ref[...] * x3_ref[...].astype(jnp.float32) + c_ref[...], 0.0)
    o_ref[...] = jnp.max(y3.reshape(BLK // K, K, 256), axis=1)


def _affine(st, g, be):
    m = st[0] / ROWS
    v = st[1] / ROWS - m * m
    a = g / jnp.sqrt(v + 1e-5)
    return a[None, :], (be - a * m)[None, :]


def _row_spec(cols):
    return pl.BlockSpec((BLK, cols), lambda i: (i, 0))


def _full_spec(r, c):
    return pl.BlockSpec((r, c), lambda i: (0, 0))


def _m1(gxn, gp, wx, wp, b1):
    return pl.pallas_call(
        _m1_body,
        grid=(NBLK,),
        in_specs=[_row_spec(8), _row_spec(128), _full_spec(8, 128),
                  _full_spec(128, 128), _full_spec(1, 128)],
        out_specs=[_row_spec(128), _full_spec(2, 128)],
        out_shape=[jax.ShapeDtypeStruct((ROWS, 128), jnp.bfloat16),
                   jax.ShapeDtypeStruct((2, 128), jnp.float32)],
    )(gxn, gp, wx, wp, b1)


def _mid(xin, a, c, w, b, cout):
    cin = xin.shape[1]
    return pl.pallas_call(
        _mid_body,
        grid=(NBLK,),
        in_specs=[_row_spec(cin), _full_spec(1, cin), _full_spec(1, cin),
                  _full_spec(cin, cout), _full_spec(1, cout)],
        out_specs=[_row_spec(cout), _full_spec(2, cout)],
        out_shape=[jax.ShapeDtypeStruct((ROWS, cout), jnp.bfloat16),
                   jax.ShapeDtypeStruct((2, cout), jnp.float32)],
    )(xin, a, c, w, b)


def _m4(x3, a, c):
    return pl.pallas_call(
        _m4_body,
        grid=(NBLK,),
        in_specs=[_row_spec(256), _full_spec(1, 256), _full_spec(1, 256)],
        out_specs=pl.BlockSpec((BLK // K, 256), lambda i: (i, 0)),
        out_shape=jax.ShapeDtypeStruct((B * S, 256), jnp.float32),
    )(x3, a, c)


def kernel(xyz, points, W1, b1, g1, be1, W2, b2, g2, be2, W3, b3, g3, be3):
    xt = jnp.transpose(xyz, (2, 0, 1))
    f0 = jax.random.randint(jax.random.key(42), (B,), 0, N, jnp.int32)[:, None]
    nx, ny, nz = _fps(xt, f0)
    nxyz_flat = jnp.concatenate(
        [nx.T[:, None, :], ny.T[:, None, :], nz.T[:, None, :]], 1).reshape(-1)
    new_xyz = jnp.stack([nx.T, ny.T, nz.T], -1)

    nxyzp = jnp.pad(new_xyz, ((0, 0), (0, 0), (0, 5)))
    xyzp = jnp.pad(jnp.transpose(xyz, (0, 2, 1)), ((0, 0), (0, 5), (0, 0)))
    dsq = _bq(nxyzp, xyzp).reshape(-1)

    pts2 = jnp.pad(points.reshape(B * N, D), ((0, 0), (0, 128 - D)))
    gxn_flat, gp = _sc_group(
        xyz[:, :, 0].reshape(-1), xyz[:, :, 1].reshape(-1),
        xyz[:, :, 2].reshape(-1), nxyz_flat, dsq, pts2)
    gxn = gxn_flat.reshape(ROWS, 8)

    wx = jnp.pad(W1[:, :3].T, ((0, 5), (0, 0)))
    wp = jnp.pad(W1[:, 3:].T, ((0, 128 - D), (0, 0)))
    x1, st1 = _m1(gxn, gp, wx, wp, b1[None, :])
    a1, c1 = _affine(st1, g1, be1)
    x2, st2 = _mid(x1, a1, c1, W2.T, b2[None, :], 128)
    a2, c2 = _affine(st2, g2, be2)
    x3, st3 = _mid(x2, a2, c2, W3.T, b3[None, :], 256)
    a3, c3 = _affine(st3, g3, be3)
    out = _m4(x3, a3, c3)

    return new_xyz, out.reshape(B, S, 256)


# speedup vs baseline: 14.1139x; 1.1285x over previous
import dataclasses
import functools

import jax
import jax.numpy as jnp
from jax import lax
from jax.experimental import pallas as pl
from jax.experimental.pallas import tpu as pltpu
from jax.experimental.pallas import tpu_sc as plsc

B, N, S, K, D = 16, 2048, 512, 32, 64
R2 = 0.2 * 0.2
ROWS = B * S * K
BLK = 4096
NBLK = ROWS // BLK


def _fps_body(xt_ref, f0_ref, nx_ref, ny_ref, nz_ref, dist_ref):
    lane = lax.broadcasted_iota(jnp.int32, (B, N), 1)
    x = xt_ref[0]
    y = xt_ref[1]
    z = xt_ref[2]
    dist_ref[...] = jnp.full((B, N), 1e10, jnp.float32)
    sub = lax.broadcasted_iota(jnp.int32, (B, B), 0)
    ln2 = lax.broadcasted_iota(jnp.int32, (B, B), 1)
    eye_b = sub == ln2

    def step(s, fa):
        oh = lane == fa
        cx = jnp.sum(jnp.where(oh, x, 0.0), 1, keepdims=True)
        cy = jnp.sum(jnp.where(oh, y, 0.0), 1, keepdims=True)
        cz = jnp.sum(jnp.where(oh, z, 0.0), 1, keepdims=True)
        nx_ref[pl.ds(s, 1), :] = jnp.sum(jnp.where(eye_b, cx, 0.0), 0, keepdims=True)
        ny_ref[pl.ds(s, 1), :] = jnp.sum(jnp.where(eye_b, cy, 0.0), 0, keepdims=True)
        nz_ref[pl.ds(s, 1), :] = jnp.sum(jnp.where(eye_b, cz, 0.0), 0, keepdims=True)
        d = (x - cx) ** 2 + (y - cy) ** 2 + (z - cz) ** 2
        dist = jnp.minimum(dist_ref[...], d)
        dist_ref[...] = dist
        m = jnp.max(dist, 1, keepdims=True)
        return jnp.min(jnp.where(dist == m, lane, N), 1, keepdims=True)

    lax.fori_loop(0, S, step, f0_ref[...])


def _fps(xt, f0):
    return pl.pallas_call(
        _fps_body,
        out_shape=[jax.ShapeDtypeStruct((S, B), jnp.float32)] * 3,
        scratch_shapes=[pltpu.VMEM((B, N), jnp.float32)],
    )(xt, f0)


def _bq_body(nxyz_ref, xyzp_ref, sq_ref):
    nb = nxyz_ref[0]
    xp = xyzp_ref[0]
    n2s = (nb[:, 0:1] * nb[:, 0:1] + nb[:, 1:2] * nb[:, 1:2]
           + nb[:, 2:3] * nb[:, 2:3])
    n2p = (xp[0:1, :] * xp[0:1, :] + xp[1:2, :] * xp[1:2, :]
           + xp[2:3, :] * xp[2:3, :])
    dots = jnp.dot(nb, xp, preferred_element_type=jnp.float32)
    sq_ref[0] = (n2s + n2p) - 2.0 * dots


def _bq(nxyzp, xyzp):
    return pl.pallas_call(
        _bq_body,
        grid=(B,),
        in_specs=[pl.BlockSpec((1, S, 8), lambda b: (b, 0, 0)),
                  pl.BlockSpec((1, 8, N), lambda b: (b, 0, 0))],
        out_specs=pl.BlockSpec((1, S, N), lambda b: (b, 0, 0)),
        out_shape=jax.ShapeDtypeStruct((B, S, N), jnp.float32),
    )(nxyzp, xyzp)


GB = 8
NBLOCKS = 256 // GB


def _sc_body(xs_hbm, ys_hbm, zs_hbm, nxyz_hbm, dsq_hbm, pts_hbm,
             gxn_hbm, gp_hbm,
             x_v, y_v, z_v, nx_v, ny_v, nz_v, dv0_v, dv1_v,
             ibuf_v, pibuf_v, gxn0_v, gxn1_v, gp0_v, gp1_v,
             sem, dsem, gsem, xsem, psem):
    wid = lax.axis_index("c") * 16 + lax.axis_index("s")
    b = wid // 2
    off = (wid % 2) * 256
    iota = lax.broadcasted_iota(jnp.int32, (16,), 0)

    pltpu.async_copy(xs_hbm.at[pl.ds(b * N, N)], x_v, sem).wait()
    pltpu.async_copy(ys_hbm.at[pl.ds(b * N, N)], y_v, sem).wait()
    pltpu.async_copy(zs_hbm.at[pl.ds(b * N, N)], z_v, sem).wait()
    pltpu.async_copy(nxyz_hbm.at[pl.ds((b * 3 + 0) * S + off, 256)], nx_v, sem).wait()
    pltpu.async_copy(nxyz_hbm.at[pl.ds((b * 3 + 1) * S + off, 256)], ny_v, sem).wait()
    pltpu.async_copy(nxyz_hbm.at[pl.ds((b * 3 + 2) * S + off, 256)], nz_v, sem).wait()

    @pl.loop(0, GB * K * 8 // 16)
    def _z(i):
        gxn0_v[pl.ds(i * 16, 16)] = jnp.zeros((16,), jnp.float32)
        gxn1_v[pl.ds(i * 16, 16)] = jnp.zeros((16,), jnp.float32)

    srow0 = b * S + off
    DB = GB * N

    def wait_dv(dv):
        pltpu.make_async_copy(dsq_hbm.at[pl.ds(0, DB)], dv, dsem).wait()

    def process(bb, dv, gxn_v, gp_v):
        @pl.when(bb >= 2)
        def _():
            pltpu.make_async_copy(gxn_v, gxn_hbm.at[pl.ds(0, GB * K * 8)],
                                  xsem).wait()
            pltpu.make_async_copy(gp_v, gp_hbm.at[pl.ds(0, GB * K)],
                                  psem).wait()
        for j in range(GB):
            cidx = bb * GB + j
            sel = (iota == (cidx % 16)).astype(jnp.float32)
            jc = (cidx // 16) * 16
            cx = jnp.sum(sel * nx_v[pl.ds(jc, 16)])
            cy = jnp.sum(sel * ny_v[pl.ds(jc, 16)])
            cz = jnp.sum(sel * nz_v[pl.ds(jc, 16)])

            def cond(st):
                return (st[1] < K) & (st[0] < N // 16)

            def chunk(st):
                c, cnt = st
                d0 = dv[pl.ds(j * N + c * 16, 16)]
                m0 = d0 <= R2
                plsc.store_compressed(ibuf_v.at[pl.ds(cnt, 16)],
                                      c * 16 + iota, mask=m0)
                cnt = cnt + plsc.all_reduce_population_count(m0)[0]
                d1 = dv[pl.ds(j * N + c * 16 + 16, 16)]
                m1 = d1 <= R2
                plsc.store_compressed(ibuf_v.at[pl.ds(cnt, 16)],
                                      c * 16 + 16 + iota, mask=m1)
                cnt = cnt + plsc.all_reduce_population_count(m1)[0]
                return c + 2, cnt

            _, cnt = lax.while_loop(cond, chunk, (jnp.int32(0), jnp.int32(0)))

            v0 = ibuf_v[pl.ds(0, 16)]
            fidx = jnp.sum(jnp.where(iota == 0, v0, 0))
            for half in (0, 1):
                slot = iota + 16 * half
                cur = ibuf_v[pl.ds(16 * half, 16)]
                ibuf_v[pl.ds(16 * half, 16)] = jnp.where(slot < cnt, cur, fidx)

            for half in (0, 1):
                giv = ibuf_v[pl.ds(16 * half, 16)]
                gx = plsc.load_gather(x_v, [giv]) - cx
                gy = plsc.load_gather(y_v, [giv]) - cy
                gz = plsc.load_gather(z_v, [giv]) - cz
                posb = (j * K + 16 * half + iota) * 8
                plsc.store_scatter(gxn_v, [posb], gx)
                plsc.store_scatter(gxn_v, [posb + 1], gy)
                plsc.store_scatter(gxn_v, [posb + 2], gz)
                pibuf_v[pl.ds(j * K + 16 * half, 16)] = giv + b * N

        row0 = (srow0 + bb * GB) * K
        h1 = pltpu.async_copy(pts_hbm.at[pibuf_v.at[pl.ds(0, 128)]],
                              gp_v.at[pl.ds(0, 128)], gsem)
        h2 = pltpu.async_copy(pts_hbm.at[pibuf_v.at[pl.ds(128, 128)]],
                              gp_v.at[pl.ds(128, 128)], gsem)
        pltpu.async_copy(gxn_v, gxn_hbm.at[pl.ds(row0 * 8, GB * K * 8)], xsem)
        h1.wait()
        h2.wait()
        pltpu.async_copy(gp_v, gp_hbm.at[pl.ds(row0, GB * K)], psem)

    pltpu.async_copy(dsq_hbm.at[pl.ds(srow0 * N, DB)], dv0_v, dsem)

    @pl.loop(0, NBLOCKS, step=2)
    def _blk(bi):
        pltpu.async_copy(dsq_hbm.at[pl.ds((srow0 + (bi + 1) * GB) * N, DB)],
                         dv1_v, dsem)
        wait_dv(dv0_v)
        process(bi, dv0_v, gxn0_v, gp0_v)

        @pl.when(bi + 2 < NBLOCKS)
        def _():
            pltpu.async_copy(dsq_hbm.at[pl.ds((srow0 + (bi + 2) * GB) * N, DB)],
                             dv0_v, dsem)

        wait_dv(dv1_v)
        process(bi + 1, dv1_v, gxn1_v, gp1_v)

    for gxn_v, gp_v in ((gxn0_v, gp0_v), (gxn1_v, gp1_v)):
        pltpu.make_async_copy(gxn_v, gxn_hbm.at[pl.ds(0, GB * K * 8)], xsem).wait()
        pltpu.make_async_copy(gp_v, gp_hbm.at[pl.ds(0, GB * K)], psem).wait()


def _sc_group(xs, ys, zs, nxyz_flat, dsq_flat, pts2):
    mesh = plsc.VectorSubcoreMesh(core_axis_name="c", subcore_axis_name="s")
    cp = pltpu.CompilerParams()
    if "needs_layout_passes" in pltpu.CompilerParams.__dataclass_fields__:
        cp = dataclasses.replace(cp, needs_layout_passes=False)
    kern = pl.kernel(
        _sc_body,
        compiler_params=cp,
        out_type=(
            jax.ShapeDtypeStruct((ROWS * 8,), jnp.float32),
            jax.ShapeDtypeStruct((ROWS, 128), jnp.float32),
        ),
        mesh=mesh,
        scratch_types=[
            pltpu.VMEM((N,), jnp.float32),
            pltpu.VMEM((N,), jnp.float32),
            pltpu.VMEM((N,), jnp.float32),
            pltpu.VMEM((256,), jnp.float32),
            pltpu.VMEM((256,), jnp.float32),
            pltpu.VMEM((256,), jnp.float32),
            pltpu.VMEM((GB * N,), jnp.float32),
            pltpu.VMEM((GB * N,), jnp.float32),
            pltpu.VMEM((96,), jnp.int32),
            pltpu.VMEM((GB * K,), jnp.int32),
            pltpu.VMEM((GB * K * 8,), jnp.float32),
            pltpu.VMEM((GB * K * 8,), jnp.float32),
            pltpu.VMEM((GB * K, 128), jnp.float32),
            pltpu.VMEM((GB * K, 128), jnp.float32),
            pltpu.SemaphoreType.DMA,
            pltpu.SemaphoreType.DMA,
            pltpu.SemaphoreType.DMA,
            pltpu.SemaphoreType.DMA,
            pltpu.SemaphoreType.DMA,
        ],
    )
    return kern(xs, ys, zs, nxyz_flat, dsq_flat, pts2)


def _m1_body(gx_ref, gp_ref, wx_ref, wp_ref, b_ref, x1_ref, st_ref):
    i = pl.program_id(0)
    x1 = (jnp.dot(gx_ref[...], wx_ref[...], preferred_element_type=jnp.float32)
          + jnp.dot(gp_ref[...], wp_ref[...], preferred_element_type=jnp.float32)
          + b_ref[...])
    x1_ref[...] = x1.astype(jnp.bfloat16)

    @pl.when(i == 0)
    def _():
        st_ref[...] = jnp.zeros_like(st_ref)

    st_ref[0:1, :] += jnp.sum(x1, 0, keepdims=True)
    st_ref[1:2, :] += jnp.sum(x1 * x1, 0, keepdims=True)


def _mid_body(xin_ref, a_ref, c_ref, w_ref, b_ref, xo_ref, st_ref):
    i = pl.program_id(0)
    xin = xin_ref[...].astype(jnp.float32)
    yprev = jnp.maximum(a_ref[...] * xin + c_ref[...], 0.0)
    xo = jnp.dot(yprev, w_ref[...], preferred_element_type=jnp.float32) + b_ref[...]
    xo_ref[...] = xo.astype(jnp.bfloat16)

    @pl.when(i == 0)
    def _():
        st_ref[...] = jnp.zeros_like(st_ref)

    st_ref[0:1, :] += jnp.sum(xo, 0, keepdims=True)
    st_ref[1:2, :] += jnp.sum(xo * xo, 0, keepdims=True)


def _m4_body(x3_ref, a_ref, c_ref, o_ref):
    y3 = jnp.maximum(a_ref[...] * x3_ref[...].astype(jnp.float32) + c_ref[...], 0.0)
    o_ref[...] = jnp.max(y3.reshape(BLK // K, K, 256), axis=1)


def _affine(st, g, be):
    m = st[0] / ROWS
    v = st[1] / ROWS - m * m
    a = g / jnp.sqrt(v + 1e-5)
    return a[None, :], (be - a * m)[None, :]


def _row_spec(cols):
    return pl.BlockSpec((BLK, cols), lambda i: (i, 0))


def _full_spec(r, c):
    return pl.BlockSpec((r, c), lambda i: (0, 0))


def _m1(gxn, gp, wx, wp, b1):
    return pl.pallas_call(
        _m1_body,
        grid=(NBLK,),
        in_specs=[_row_spec(8), _row_spec(128), _full_spec(8, 128),
                  _full_spec(128, 128), _full_spec(1, 128)],
        out_specs=[_row_spec(128), _full_spec(2, 128)],
        out_shape=[jax.ShapeDtypeStruct((ROWS, 128), jnp.bfloat16),
                   jax.ShapeDtypeStruct((2, 128), jnp.float32)],
    )(gxn, gp, wx, wp, b1)


def _mid(xin, a, c, w, b, cout):
    cin = xin.shape[1]
    return pl.pallas_call(
        _mid_body,
        grid=(NBLK,),
        in_specs=[_row_spec(cin), _full_spec(1, cin), _full_spec(1, cin),
                  _full_spec(cin, cout), _full_spec(1, cout)],
        out_specs=[_row_spec(cout), _full_spec(2, cout)],
        out_shape=[jax.ShapeDtypeStruct((ROWS, cout), jnp.bfloat16),
                   jax.ShapeDtypeStruct((2, cout), jnp.float32)],
    )(xin, a, c, w, b)


def _m4(x3, a, c):
    return pl.pallas_call(
        _m4_body,
        grid=(NBLK,),
        in_specs=[_row_spec(256), _full_spec(1, 256), _full_spec(1, 256)],
        out_specs=pl.BlockSpec((BLK // K, 256), lambda i: (i, 0)),
        out_shape=jax.ShapeDtypeStruct((B * S, 256), jnp.float32),
    )(x3, a, c)


def kernel(xyz, points, W1, b1, g1, be1, W2, b2, g2, be2, W3, b3, g3, be3):
    xt = jnp.transpose(xyz, (2, 0, 1))
    f0 = jax.random.randint(jax.random.key(42), (B,), 0, N, jnp.int32)[:, None]
    nx, ny, nz = _fps(xt, f0)
    nxyz_flat = jnp.concatenate(
        [nx.T[:, None, :], ny.T[:, None, :], nz.T[:, None, :]], 1).reshape(-1)
    new_xyz = jnp.stack([nx.T, ny.T, nz.T], -1)

    nxyzp = jnp.pad(new_xyz, ((0, 0), (0, 0), (0, 5)))
    xyzp = jnp.pad(jnp.transpose(xyz, (0, 2, 1)), ((0, 0), (0, 5), (0, 0)))
    dsq = _bq(nxyzp, xyzp).reshape(-1)

    pts2 = jnp.pad(points.reshape(B * N, D), ((0, 0), (0, 128 - D)))
    gxn_flat, gp = _sc_group(
        xyz[:, :, 0].reshape(-1), xyz[:, :, 1].reshape(-1),
        xyz[:, :, 2].reshape(-1), nxyz_flat, dsq, pts2)
    gxn = gxn_flat.reshape(ROWS, 8)

    wx = jnp.pad(W1[:, :3].T, ((0, 5), (0, 0)))
    wp = jnp.pad(W1[:, 3:].T, ((0, 128 - D), (0, 0)))
    x1, st1 = _m1(gxn, gp, wx, wp, b1[None, :])
    a1, c1 = _affine(st1, g1, be1)
    x2, st2 = _mid(x1, a1, c1, W2.T, b2[None, :], 128)
    a2, c2 = _affine(st2, g2, be2)
    x3, st3 = _mid(x2, a2, c2, W3.T, b3[None, :], 256)
    a3, c3 = _affine(st3, g3, be3)
    out = _m4(x3, a3, c3)

    return new_xyz, out.reshape(B, S, 256)


# speedup vs baseline: 14.8759x; 1.0540x over previous
import dataclasses
import functools

import jax
import jax.numpy as jnp
from jax import lax
from jax.experimental import pallas as pl
from jax.experimental.pallas import tpu as pltpu
from jax.experimental.pallas import tpu_sc as plsc

B, N, S, K, D = 16, 2048, 512, 32, 64
R2 = 0.2 * 0.2
ROWS = B * S * K
BLK = 8192
NBLK = ROWS // BLK


def _fps_body(xt_ref, f0_ref, nx_ref, ny_ref, nz_ref, dist_ref):
    lane = lax.broadcasted_iota(jnp.int32, (B, N), 1)
    x = xt_ref[0]
    y = xt_ref[1]
    z = xt_ref[2]
    dist_ref[...] = jnp.full((B, N), 1e10, jnp.float32)
    sub = lax.broadcasted_iota(jnp.int32, (B, B), 0)
    ln2 = lax.broadcasted_iota(jnp.int32, (B, B), 1)
    eye_b = sub == ln2

    def step(s, fa):
        oh = lane == fa
        cx = jnp.sum(jnp.where(oh, x, 0.0), 1, keepdims=True)
        cy = jnp.sum(jnp.where(oh, y, 0.0), 1, keepdims=True)
        cz = jnp.sum(jnp.where(oh, z, 0.0), 1, keepdims=True)
        nx_ref[pl.ds(s, 1), :] = jnp.sum(jnp.where(eye_b, cx, 0.0), 0, keepdims=True)
        ny_ref[pl.ds(s, 1), :] = jnp.sum(jnp.where(eye_b, cy, 0.0), 0, keepdims=True)
        nz_ref[pl.ds(s, 1), :] = jnp.sum(jnp.where(eye_b, cz, 0.0), 0, keepdims=True)
        d = (x - cx) ** 2 + (y - cy) ** 2 + (z - cz) ** 2
        dist = jnp.minimum(dist_ref[...], d)
        dist_ref[...] = dist
        m = jnp.max(dist, 1, keepdims=True)
        return jnp.min(jnp.where(dist == m, lane, N), 1, keepdims=True)

    lax.fori_loop(0, S, step, f0_ref[...])


def _fps(xt, f0):
    return pl.pallas_call(
        _fps_body,
        out_shape=[jax.ShapeDtypeStruct((S, B), jnp.float32)] * 3,
        scratch_shapes=[pltpu.VMEM((B, N), jnp.float32)],
    )(xt, f0)


def _bq_body(nxyz_ref, xyzp_ref, sq_ref):
    nb = nxyz_ref[0]
    xp = xyzp_ref[0]
    n2s = (nb[:, 0:1] * nb[:, 0:1] + nb[:, 1:2] * nb[:, 1:2]
           + nb[:, 2:3] * nb[:, 2:3])
    n2p = (xp[0:1, :] * xp[0:1, :] + xp[1:2, :] * xp[1:2, :]
           + xp[2:3, :] * xp[2:3, :])
    dots = jnp.dot(nb, xp, preferred_element_type=jnp.float32)
    sq_ref[0] = (n2s + n2p) - 2.0 * dots


def _bq(nxyzp, xyzp):
    return pl.pallas_call(
        _bq_body,
        grid=(B,),
        in_specs=[pl.BlockSpec((1, S, 8), lambda b: (b, 0, 0)),
                  pl.BlockSpec((1, 8, N), lambda b: (b, 0, 0))],
        out_specs=pl.BlockSpec((1, S, N), lambda b: (b, 0, 0)),
        out_shape=jax.ShapeDtypeStruct((B, S, N), jnp.float32),
    )(nxyzp, xyzp)


GB = 8
NBLOCKS = 256 // GB


def _sc_body(xs_hbm, ys_hbm, zs_hbm, nxyz_hbm, dsq_hbm, pts_hbm,
             gxn_hbm, gp_hbm,
             x_v, y_v, z_v, nx_v, ny_v, nz_v, dv0_v, dv1_v,
             ibuf_v, pibuf_v, gxn0_v, gxn1_v, gp0_v, gp1_v,
             sem, dsem, gsem, xsem, psem):
    wid = lax.axis_index("c") * 16 + lax.axis_index("s")
    b = wid // 2
    off = (wid % 2) * 256
    iota = lax.broadcasted_iota(jnp.int32, (16,), 0)

    pltpu.async_copy(xs_hbm.at[pl.ds(b * N, N)], x_v, sem).wait()
    pltpu.async_copy(ys_hbm.at[pl.ds(b * N, N)], y_v, sem).wait()
    pltpu.async_copy(zs_hbm.at[pl.ds(b * N, N)], z_v, sem).wait()
    pltpu.async_copy(nxyz_hbm.at[pl.ds((b * 3 + 0) * S + off, 256)], nx_v, sem).wait()
    pltpu.async_copy(nxyz_hbm.at[pl.ds((b * 3 + 1) * S + off, 256)], ny_v, sem).wait()
    pltpu.async_copy(nxyz_hbm.at[pl.ds((b * 3 + 2) * S + off, 256)], nz_v, sem).wait()

    @pl.loop(0, GB * K * 8 // 16)
    def _z(i):
        gxn0_v[pl.ds(i * 16, 16)] = jnp.zeros((16,), jnp.float32)
        gxn1_v[pl.ds(i * 16, 16)] = jnp.zeros((16,), jnp.float32)

    srow0 = b * S + off
    DB = GB * N

    def wait_dv(dv):
        pltpu.make_async_copy(dsq_hbm.at[pl.ds(0, DB)], dv, dsem).wait()

    def process(bb, dv, gxn_v, gp_v):
        @pl.when(bb >= 2)
        def _():
            pltpu.make_async_copy(gxn_v, gxn_hbm.at[pl.ds(0, GB * K * 8)],
                                  xsem).wait()
            pltpu.make_async_copy(gp_v, gp_hbm.at[pl.ds(0, GB * K)],
                                  psem).wait()
        for j in range(GB):
            cidx = bb * GB + j
            sel = (iota == (cidx % 16)).astype(jnp.float32)
            jc = (cidx // 16) * 16
            cx = jnp.sum(sel * nx_v[pl.ds(jc, 16)])
            cy = jnp.sum(sel * ny_v[pl.ds(jc, 16)])
            cz = jnp.sum(sel * nz_v[pl.ds(jc, 16)])

            def cond(st):
                return (st[1] < K) & (st[0] < N // 16)

            def chunk(st):
                c, cnt = st
                d0 = dv[pl.ds(j * N + c * 16, 16)]
                m0 = d0 <= R2
                plsc.store_compressed(ibuf_v.at[pl.ds(cnt, 16)],
                                      c * 16 + iota, mask=m0)
                cnt = cnt + plsc.all_reduce_population_count(m0)[0]
                d1 = dv[pl.ds(j * N + c * 16 + 16, 16)]
                m1 = d1 <= R2
                plsc.store_compressed(ibuf_v.at[pl.ds(cnt, 16)],
                                      c * 16 + 16 + iota, mask=m1)
                cnt = cnt + plsc.all_reduce_population_count(m1)[0]
                return c + 2, cnt

            _, cnt = lax.while_loop(cond, chunk, (jnp.int32(0), jnp.int32(0)))

            v0 = ibuf_v[pl.ds(0, 16)]
            fidx = jnp.sum(jnp.where(iota == 0, v0, 0))
            for half in (0, 1):
                slot = iota + 16 * half
                cur = ibuf_v[pl.ds(16 * half, 16)]
                ibuf_v[pl.ds(16 * half, 16)] = jnp.where(slot < cnt, cur, fidx)

            for half in (0, 1):
                giv = ibuf_v[pl.ds(16 * half, 16)]
                gx = plsc.load_gather(x_v, [giv]) - cx
                gy = plsc.load_gather(y_v, [giv]) - cy
                gz = plsc.load_gather(z_v, [giv]) - cz
                posb = (j * K + 16 * half + iota) * 8
                plsc.store_scatter(gxn_v, [posb], gx)
                plsc.store_scatter(gxn_v, [posb + 1], gy)
                plsc.store_scatter(gxn_v, [posb + 2], gz)
                pibuf_v[pl.ds(j * K + 16 * half, 16)] = giv + b * N

        row0 = (srow0 + bb * GB) * K
        h1 = pltpu.async_copy(pts_hbm.at[pibuf_v.at[pl.ds(0, 128)]],
                              gp_v.at[pl.ds(0, 128)], gsem)
        h2 = pltpu.async_copy(pts_hbm.at[pibuf_v.at[pl.ds(128, 128)]],
                              gp_v.at[pl.ds(128, 128)], gsem)
        pltpu.async_copy(gxn_v, gxn_hbm.at[pl.ds(row0 * 8, GB * K * 8)], xsem)
        h1.wait()
        h2.wait()
        pltpu.async_copy(gp_v, gp_hbm.at[pl.ds(row0, GB * K)], psem)

    pltpu.async_copy(dsq_hbm.at[pl.ds(srow0 * N, DB)], dv0_v, dsem)

    @pl.loop(0, NBLOCKS, step=2)
    def _blk(bi):
        pltpu.async_copy(dsq_hbm.at[pl.ds((srow0 + (bi + 1) * GB) * N, DB)],
                         dv1_v, dsem)
        wait_dv(dv0_v)
        process(bi, dv0_v, gxn0_v, gp0_v)

        @pl.when(bi + 2 < NBLOCKS)
        def _():
            pltpu.async_copy(dsq_hbm.at[pl.ds((srow0 + (bi + 2) * GB) * N, DB)],
                             dv0_v, dsem)

        wait_dv(dv1_v)
        process(bi + 1, dv1_v, gxn1_v, gp1_v)

    for gxn_v, gp_v in ((gxn0_v, gp0_v), (gxn1_v, gp1_v)):
        pltpu.make_async_copy(gxn_v, gxn_hbm.at[pl.ds(0, GB * K * 8)], xsem).wait()
        pltpu.make_async_copy(gp_v, gp_hbm.at[pl.ds(0, GB * K)], psem).wait()


def _sc_group(xs, ys, zs, nxyz_flat, dsq_flat, pts2):
    mesh = plsc.VectorSubcoreMesh(core_axis_name="c", subcore_axis_name="s")
    cp = pltpu.CompilerParams()
    if "needs_layout_passes" in pltpu.CompilerParams.__dataclass_fields__:
        cp = dataclasses.replace(cp, needs_layout_passes=False)
    kern = pl.kernel(
        _sc_body,
        compiler_params=cp,
        out_type=(
            jax.ShapeDtypeStruct((ROWS * 8,), jnp.float32),
            jax.ShapeDtypeStruct((ROWS, 128), jnp.float32),
        ),
        mesh=mesh,
        scratch_types=[
            pltpu.VMEM((N,), jnp.float32),
            pltpu.VMEM((N,), jnp.float32),
            pltpu.VMEM((N,), jnp.float32),
            pltpu.VMEM((256,), jnp.float32),
            pltpu.VMEM((256,), jnp.float32),
            pltpu.VMEM((256,), jnp.float32),
            pltpu.VMEM((GB * N,), jnp.float32),
            pltpu.VMEM((GB * N,), jnp.float32),
            pltpu.VMEM((96,), jnp.int32),
            pltpu.VMEM((GB * K,), jnp.int32),
            pltpu.VMEM((GB * K * 8,), jnp.float32),
            pltpu.VMEM((GB * K * 8,), jnp.float32),
            pltpu.VMEM((GB * K, 128), jnp.float32),
            pltpu.VMEM((GB * K, 128), jnp.float32),
            pltpu.SemaphoreType.DMA,
            pltpu.SemaphoreType.DMA,
            pltpu.SemaphoreType.DMA,
            pltpu.SemaphoreType.DMA,
            pltpu.SemaphoreType.DMA,
        ],
    )
    return kern(xs, ys, zs, nxyz_flat, dsq_flat, pts2)


def _m1_body(gx_ref, gp_ref, wx_ref, wp_ref, b_ref, x1_ref, st_ref):
    i = pl.program_id(0)
    x1 = (jnp.dot(gx_ref[...], wx_ref[...], preferred_element_type=jnp.float32)
          + jnp.dot(gp_ref[...], wp_ref[...], preferred_element_type=jnp.float32)
          + b_ref[...])
    x1_ref[...] = x1.astype(jnp.bfloat16)

    @pl.when(i == 0)
    def _():
        st_ref[...] = jnp.zeros_like(st_ref)

    st_ref[0:1, :] += jnp.sum(x1, 0, keepdims=True)
    st_ref[1:2, :] += jnp.sum(x1 * x1, 0, keepdims=True)


def _mid_body(xin_ref, a_ref, c_ref, w_ref, b_ref, xo_ref, st_ref):
    i = pl.program_id(0)
    xin = xin_ref[...].astype(jnp.float32)
    yprev = jnp.maximum(a_ref[...] * xin + c_ref[...], 0.0)
    xo = jnp.dot(yprev, w_ref[...], preferred_element_type=jnp.float32) + b_ref[...]
    xo_ref[...] = xo.astype(jnp.bfloat16)

    @pl.when(i == 0)
    def _():
        st_ref[...] = jnp.zeros_like(st_ref)

    st_ref[0:1, :] += jnp.sum(xo, 0, keepdims=True)
    st_ref[1:2, :] += jnp.sum(xo * xo, 0, keepdims=True)


def _m4_body(x3_ref, a_ref, c_ref, o_ref):
    y3 = jnp.maximum(a_ref[...] * x3_ref[...].astype(jnp.float32) + c_ref[...], 0.0)
    o_ref[...] = jnp.max(y3.reshape(BLK // K, K, 256), axis=1)


def _affine(st, g, be):
    m = st[0] / ROWS
    v = st[1] / ROWS - m * m
    a = g / jnp.sqrt(v + 1e-5)
    return a[None, :], (be - a * m)[None, :]


def _row_spec(cols):
    return pl.BlockSpec((BLK, cols), lambda i: (i, 0))


def _full_spec(r, c):
    return pl.BlockSpec((r, c), lambda i: (0, 0))


def _m1(gxn, gp, wx, wp, b1):
    return pl.pallas_call(
        _m1_body,
        grid=(NBLK,),
        in_specs=[_row_spec(8), _row_spec(128), _full_spec(8, 128),
                  _full_spec(128, 128), _full_spec(1, 128)],
        out_specs=[_row_spec(128), _full_spec(2, 128)],
        out_shape=[jax.ShapeDtypeStruct((ROWS, 128), jnp.bfloat16),
                   jax.ShapeDtypeStruct((2, 128), jnp.float32)],
    )(gxn, gp, wx, wp, b1)


def _mid(xin, a, c, w, b, cout):
    cin = xin.shape[1]
    return pl.pallas_call(
        _mid_body,
        grid=(NBLK,),
        in_specs=[_row_spec(cin), _full_spec(1, cin), _full_spec(1, cin),
                  _full_spec(cin, cout), _full_spec(1, cout)],
        out_specs=[_row_spec(cout), _full_spec(2, cout)],
        out_shape=[jax.ShapeDtypeStruct((ROWS, cout), jnp.bfloat16),
                   jax.ShapeDtypeStruct((2, cout), jnp.float32)],
    )(xin, a, c, w, b)


def _m4(x3, a, c):
    return pl.pallas_call(
        _m4_body,
        grid=(NBLK,),
        in_specs=[_row_spec(256), _full_spec(1, 256), _full_spec(1, 256)],
        out_specs=pl.BlockSpec((BLK // K, 256), lambda i: (i, 0)),
        out_shape=jax.ShapeDtypeStruct((B * S, 256), jnp.float32),
    )(x3, a, c)


def kernel(xyz, points, W1, b1, g1, be1, W2, b2, g2, be2, W3, b3, g3, be3):
    xt = jnp.transpose(xyz, (2, 0, 1))
    f0 = jax.random.randint(jax.random.key(42), (B,), 0, N, jnp.int32)[:, None]
    nx, ny, nz = _fps(xt, f0)
    nxyz_flat = jnp.concatenate(
        [nx.T[:, None, :], ny.T[:, None, :], nz.T[:, None, :]], 1).reshape(-1)
    new_xyz = jnp.stack([nx.T, ny.T, nz.T], -1)

    nxyzp = jnp.pad(new_xyz, ((0, 0), (0, 0), (0, 5)))
    xyzp = jnp.pad(jnp.transpose(xyz, (0, 2, 1)), ((0, 0), (0, 5), (0, 0)))
    dsq = _bq(nxyzp, xyzp).reshape(-1)

    pts2 = jnp.pad(points.reshape(B * N, D), ((0, 0), (0, 128 - D)))
    gxn_flat, gp = _sc_group(
        xyz[:, :, 0].reshape(-1), xyz[:, :, 1].reshape(-1),
        xyz[:, :, 2].reshape(-1), nxyz_flat, dsq, pts2)
    gxn = gxn_flat.reshape(ROWS, 8)

    wx = jnp.pad(W1[:, :3].T, ((0, 5), (0, 0)))
    wp = jnp.pad(W1[:, 3:].T, ((0, 128 - D), (0, 0)))
    x1, st1 = _m1(gxn, gp, wx, wp, b1[None, :])
    a1, c1 = _affine(st1, g1, be1)
    x2, st2 = _mid(x1, a1, c1, W2.T, b2[None, :], 128)
    a2, c2 = _affine(st2, g2, be2)
    x3, st3 = _mid(x2, a2, c2, W3.T, b3[None, :], 256)
    a3, c3 = _affine(st3, g3, be3)
    out = _m4(x3, a3, c3)

    return new_xyz, out.reshape(B, S, 256)
